# baseline (speedup 1.0000x reference)
import functools

import jax
import jax.numpy as jnp
from jax import lax
from jax.experimental import pallas as pl
from jax.experimental.pallas import tpu as pltpu
from jax.experimental.pallas import tpu_sc as plsc

N = 10000
E = 320000
F = 128

NC = 2
NS = 16
NW = NC * NS
L = 16

EP = 335872
CH = 128
EW = EP // NW
CPW = EW // CH

EPW0 = E // NW
CH0 = 80
CPW0 = EPW0 // CH0

NP = 10240
RPT = NP // NS

NB = 10
BN = N // NB
BE = 2048
GE = EP // BE

def _sc_preproc_body(src_hbm, dst_hbm, ew_hbm, zn_hbm,
                     ea0_hbm, valid_hbm, cnt_hbm, eas_hbm,
                     sall, dall, wall, eaall, vall, cnt_v, eas_v):
    c = lax.axis_index("c")
    s = lax.axis_index("s")
    w = c * NS + s
    b0 = w * EPW0
    pltpu.sync_copy(zn_hbm, cnt_v)
    pltpu.sync_copy(zn_hbm, eas_v)
    pltpu.sync_copy(src_hbm.at[pl.ds(b0, EPW0)], sall)
    pltpu.sync_copy(dst_hbm.at[pl.ds(b0, EPW0)], dall)
    pltpu.sync_copy(ew_hbm.at[pl.ds(b0, EPW0)], wall)

    UNROLL = 5

    def step(i, carry):
        for j in range(UNROLL):
            sl = pl.ds((i * UNROLL + j) * L, L)
            sv = sall[sl]
            dv = dall[sl]
            wv = wall[sl]
            mk = jnp.where(sv != dv, jnp.float32(1.0), jnp.float32(0.0))
            ea = mk * wv
            eaall[sl] = ea
            vall[sl] = mk
            plsc.addupdate_scatter(cnt_v, [dv], mk)
            plsc.addupdate_scatter(eas_v, [dv], ea)
        return carry

    lax.fori_loop(0, EPW0 // (L * UNROLL), step, 0)
    pltpu.sync_copy(eaall, ea0_hbm.at[pl.ds(b0, EPW0)])
    pltpu.sync_copy(vall, valid_hbm.at[pl.ds(b0, EPW0)])
    pltpu.sync_copy(cnt_v, cnt_hbm.at[pl.ds(w * N, N)])
    pltpu.sync_copy(eas_v, eas_hbm.at[pl.ds(w * N, N)])


@functools.lru_cache(maxsize=None)
def _sc_kernels():
    mesh = plsc.VectorSubcoreMesh(
        core_axis_name="c", subcore_axis_name="s",
        num_cores=NC, num_subcores=NS,
    )
    cparams = pltpu.CompilerParams(needs_layout_passes=False)
    k0 = pl.kernel(
        _sc_preproc_body,
        out_type=[
            jax.ShapeDtypeStruct((E,), jnp.float32),
            jax.ShapeDtypeStruct((E,), jnp.float32),
            jax.ShapeDtypeStruct((NW * N,), jnp.float32),
            jax.ShapeDtypeStruct((NW * N,), jnp.float32),
        ],
        mesh=mesh,
        compiler_params=cparams,
        scratch_types=[
            pltpu.VMEM((EPW0,), jnp.int32),
            pltpu.VMEM((EPW0,), jnp.int32),
            pltpu.VMEM((EPW0,), jnp.float32),
            pltpu.VMEM((EPW0,), jnp.float32),
            pltpu.VMEM((EPW0,), jnp.float32),
            pltpu.VMEM((N,), jnp.float32),
            pltpu.VMEM((N,), jnp.float32),
        ],
    )
    k2 = pl.kernel(
        _sc_gather_body,
        out_type=[
            jax.ShapeDtypeStruct((EP, F), jnp.float32),
            jax.ShapeDtypeStruct((EP, F), jnp.float32),
        ],
        mesh=mesh,
        compiler_params=cparams,
        scratch_types=[
            pltpu.VMEM((CH,), jnp.int32),
            pltpu.VMEM((CH,), jnp.int32),
            pltpu.VMEM((CH,), jnp.int32),
            pltpu.VMEM((CH,), jnp.int32),
            pltpu.VMEM((CH, F), jnp.float32),
            pltpu.VMEM((CH, F), jnp.float32),
            pltpu.VMEM((CH, F), jnp.float32),
            pltpu.VMEM((CH, F), jnp.float32),
            pltpu.SemaphoreType.DMA,
            pltpu.SemaphoreType.DMA,
            pltpu.SemaphoreType.DMA,
            pltpu.SemaphoreType.DMA,
            pltpu.SemaphoreType.DMA,
        ],
    )
    k4 = pl.kernel(
        _sc_scatter_body,
        out_type=[
            jax.ShapeDtypeStruct((NC * NP, F), jnp.float32),
            jax.ShapeDtypeStruct((NW * N,), jnp.float32),
        ],
        mesh=mesh,
        compiler_params=cparams,
        scratch_types=[
            pltpu.VMEM((2, CH), jnp.int32),
            pltpu.VMEM((2 * CH,), jnp.int32),
            pltpu.VMEM((2 * CH,), jnp.float32),
            pltpu.VMEM((CH, F), jnp.float32),
            pltpu.VMEM((CH, F), jnp.float32),
            pltpu.VMEM((N,), jnp.float32),
            pltpu.VMEM_SHARED((NP, F), jnp.float32),
            pltpu.SemaphoreType.DMA,
            pltpu.SemaphoreType.DMA,
        ],
    )
    return k0, k2, k4


DEP = 2


def _sc_gather_body(xl_hbm, xr_hbm, src_hbm, dst_hbm, gl_hbm, gr_hbm,
                    si0, di0, si1, di1, bl0, br0, bl1, br1,
                    is0, is1, gs0, gs1, wsem):
    c = lax.axis_index("c")
    s = lax.axis_index("s")
    w = c * NS + s
    b0 = w * EW
    bufs = ((si0, di0, bl0, br0, is0, gs0), (si1, di1, bl1, br1, is1, gs1))

    def it(t, carry):
        i0 = t * DEP
        icps = []
        for k in range(DEP):
            sik, dik, blk, brk, isk, gsk = bufs[k]
            base = b0 + (i0 + k) * CH
            icps.append((
                pltpu.async_copy(src_hbm.at[pl.ds(base, CH)], sik, isk),
                pltpu.async_copy(dst_hbm.at[pl.ds(base, CH)], dik, isk),
            ))
        gcps = []
        for k in range(DEP):
            sik, dik, blk, brk, isk, gsk = bufs[k]
            icps[k][0].wait()
            icps[k][1].wait()
            gcps.append((
                pltpu.async_copy(xl_hbm.at[sik], blk, gsk),
                pltpu.async_copy(xr_hbm.at[dik], brk, gsk),
            ))
        wcps = []
        for k in range(DEP):
            gcps[k][0].wait()
            gcps[k][1].wait()
            base = b0 + (i0 + k) * CH
            wcps.append(pltpu.async_copy(bufs[k][2], gl_hbm.at[pl.ds(base, CH)], wsem))
            wcps.append(pltpu.async_copy(bufs[k][3], gr_hbm.at[pl.ds(base, CH)], wsem))
        for cp in wcps:
            cp.wait()
        return carry

    lax.fori_loop(0, CPW // DEP, it, 0)


def _sc_scatter_body(wr_hbm, ex_hbm, dst_hbm, dst3_hbm, zr_hbm, zn_hbm,
                     nump_hbm, denp_hbm,
                     di2, di1, exb, bw0, bw1, den_v, num_s,
                     ls0, ls1):
    c = lax.axis_index("c")
    s = lax.axis_index("s")
    w = c * NS + s
    r0 = s * RPT
    b0 = w * EW
    pltpu.sync_copy(zn_hbm, den_v)
    pltpu.sync_copy(zr_hbm.at[pl.ds(r0, RPT)], num_s.at[pl.ds(r0, RPT)])
    plsc.subcore_barrier()
    bufs = ((bw0, ls0), (bw1, ls1))

    def it(t, carry):
        i0 = t * DEP
        lcps = []
        for k in range(DEP):
            bwk, sk = bufs[k]
            i = i0 + k
            base = b0 + i * CH
            lcps.append((
                pltpu.async_copy(wr_hbm.at[pl.ds(base, CH)], bwk, sk),
                pltpu.async_copy(dst3_hbm.at[w, i], di2.at[k], sk),
                pltpu.async_copy(dst_hbm.at[pl.ds(base, CH)],
                                 di1.at[pl.ds(k * CH, CH)], sk),
                pltpu.async_copy(ex_hbm.at[pl.ds(base, CH)],
                                 exb.at[pl.ds(k * CH, CH)], sk),
            ))
        for k in range(DEP):
            for cp in lcps[k]:
                cp.wait()
            pltpu.sync_copy(bufs[k][0], num_s.at[di2.at[k]], add=True)
            for j in range(CH // L):
                sl = pl.ds(k * CH + j * L, L)
                plsc.addupdate_scatter(den_v, [di1[sl]], exb[sl])
        return carry

    lax.fori_loop(0, CPW // DEP, it, 0)
    plsc.subcore_barrier()
    pltpu.sync_copy(den_v, denp_hbm.at[pl.ds(w * N, N)])
    pltpu.sync_copy(num_s.at[pl.ds(r0, RPT)],
                    nump_hbm.at[pl.ds(c * NP + r0, RPT)])


def _dot(a, b):
    return jnp.dot(a, b, preferred_element_type=jnp.float32,
                   precision=lax.Precision.HIGHEST)


def _tc_prep_body(x_ref, wlt_ref, wrt_ref, bl_ref, br_ref, cnt_ref, eas_ref,
                  xl_ref, xr_ref, la_ref):
    xb = x_ref[...]
    xl_ref[...] = _dot(xb, wlt_ref[...]) + bl_ref[...]
    xr_ref[...] = _dot(xb, wrt_ref[...]) + br_ref[...]

    @pl.when(pl.program_id(0) == 0)
    def _():
        cnt = jnp.sum(cnt_ref[...], axis=0)
        eas = jnp.sum(eas_ref[...], axis=0)
        la_ref[...] = eas / jnp.maximum(cnt, 1.0)


_k1 = pl.pallas_call(
    _tc_prep_body,
    grid=(NB,),
    in_specs=[
        pl.BlockSpec((BN, F), lambda i: (i, 0)),
        pl.BlockSpec((F, F), lambda i: (0, 0)),
        pl.BlockSpec((F, F), lambda i: (0, 0)),
        pl.BlockSpec((1, F), lambda i: (0, 0)),
        pl.BlockSpec((1, F), lambda i: (0, 0)),
        pl.BlockSpec((NW, NB, BN), lambda i: (0, 0, 0)),
        pl.BlockSpec((NW, NB, BN), lambda i: (0, 0, 0)),
    ],
    out_specs=[
        pl.BlockSpec((BN, F), lambda i: (i, 0)),
        pl.BlockSpec((BN, F), lambda i: (i, 0)),
        pl.BlockSpec((NB, BN), lambda i: (0, 0)),
    ],
    out_shape=[
        jax.ShapeDtypeStruct((N, F), jnp.float32),
        jax.ShapeDtypeStruct((N, F), jnp.float32),
        jax.ShapeDtypeStruct((NB, BN), jnp.float32),
    ],
)


def _tc_edge_body(gl_ref, gr_ref, ea_ref, va_ref, we_ref, att_ref,
                  wr_ref, ex_ref):
    gl = gl_ref[...]
    m = gl + gr_ref[...] + ea_ref[...] * we_ref[...]
    m = jnp.maximum(m, 0.2 * m)
    e = jnp.sum(m * att_ref[...], axis=1, keepdims=True)
    ex = va_ref[...] * jnp.exp(e)
    ex_ref[...] = ex
    wr_ref[...] = gl * ex


_k3 = pl.pallas_call(
    _tc_edge_body,
    grid=(GE,),
    in_specs=[
        pl.BlockSpec((BE, F), lambda i: (i, 0)),
        pl.BlockSpec((BE, F), lambda i: (i, 0)),
        pl.BlockSpec((BE, 1), lambda i: (i, 0)),
        pl.BlockSpec((BE, 1), lambda i: (i, 0)),
        pl.BlockSpec((1, F), lambda i: (0, 0)),
        pl.BlockSpec((1, F), lambda i: (0, 0)),
    ],
    out_specs=[
        pl.BlockSpec((BE, F), lambda i: (i, 0)),
        pl.BlockSpec((BE, 1), lambda i: (i, 0)),
    ],
    out_shape=[
        jax.ShapeDtypeStruct((EP, F), jnp.float32),
        jax.ShapeDtypeStruct((EP, 1), jnp.float32),
    ],
)


def _tc_den_body(dp_ref, out_ref):
    out_ref[...] = jnp.sum(dp_ref[...], axis=0)


_k5a = pl.pallas_call(
    _tc_den_body,
    out_shape=jax.ShapeDtypeStruct((NB, BN), jnp.float32),
)


def _tc_comb_body(np_ref, den_ref, b_ref, wlt_ref, wrt_ref, bl_ref, br_ref,
                  xl_ref, xr_ref):
    ns = np_ref[0] + np_ref[1]
    h = ns / den_ref[...] + b_ref[...]
    h = jnp.maximum(h, 0.0)
    xl_ref[...] = _dot(h, wlt_ref[...]) + bl_ref[...]
    xr_ref[...] = _dot(h, wrt_ref[...]) + br_ref[...]


_k5 = pl.pallas_call(
    _tc_comb_body,
    grid=(NB,),
    in_specs=[
        pl.BlockSpec((NC, BN, F), lambda i: (0, i, 0)),
        pl.BlockSpec((BN, 1), lambda i: (i, 0)),
        pl.BlockSpec((1, F), lambda i: (0, 0)),
        pl.BlockSpec((F, F), lambda i: (0, 0)),
        pl.BlockSpec((F, F), lambda i: (0, 0)),
        pl.BlockSpec((1, F), lambda i: (0, 0)),
        pl.BlockSpec((1, F), lambda i: (0, 0)),
    ],
    out_specs=[
        pl.BlockSpec((BN, F), lambda i: (i, 0)),
        pl.BlockSpec((BN, F), lambda i: (i, 0)),
    ],
    out_shape=[
        jax.ShapeDtypeStruct((N, F), jnp.float32),
        jax.ShapeDtypeStruct((N, F), jnp.float32),
    ],
)


def _tc_final_body(np_ref, den_ref, b_ref, out_ref):
    out_ref[...] = (np_ref[0] + np_ref[1]) / den_ref[...] + b_ref[...]


_k5f = pl.pallas_call(
    _tc_final_body,
    grid=(NB,),
    in_specs=[
        pl.BlockSpec((NC, BN, F), lambda i: (0, i, 0)),
        pl.BlockSpec((BN, 1), lambda i: (i, 0)),
        pl.BlockSpec((1, F), lambda i: (0, 0)),
    ],
    out_specs=pl.BlockSpec((BN, F), lambda i: (i, 0)),
    out_shape=jax.ShapeDtypeStruct((N, F), jnp.float32),
)


def kernel(x, edge_index, edge_weight, Wl1, bl1, Wr1, br1, We1, att1, b1,
           Wl2, bl2, Wr2, br2, We2, att2, b2):
    src0 = edge_index[0]
    dst0 = edge_index[1]
    ew = edge_weight[:, 0]
    zn = jnp.zeros((N,), jnp.float32)
    zr = jnp.zeros((NP, F), jnp.float32)

    _k0, _k2, _k4 = _sc_kernels()
    ea0, valid0, cnt_p, eas_p = _k0(src0, dst0, ew, zn)
    xl1, xr1, la = _k1(x, Wl1.T, Wr1.T, bl1[None], br1[None],
                       cnt_p.reshape(NW, NB, BN), eas_p.reshape(NW, NB, BN))

    loop_idx = jnp.arange(N, dtype=jnp.int32)
    pad = EP - E - N
    pad_idx = jnp.arange(pad, dtype=jnp.int32) % N
    srcp = jnp.concatenate([src0, loop_idx, pad_idx])
    dstp = jnp.concatenate([dst0, loop_idx, pad_idx])
    eap = jnp.concatenate([ea0, la.reshape(N), jnp.zeros((pad,), jnp.float32)])
    vap = jnp.concatenate([valid0, jnp.ones((N,), jnp.float32),
                           jnp.zeros((pad,), jnp.float32)])
    eap2 = eap[:, None]
    vap2 = vap[:, None]

    dst3 = dstp.reshape(NW, CPW, CH)

    def layer(xl, xr, wev, attv):
        gl, gr = _k2(xl, xr, srcp, dstp)
        wrows, exc = _k3(gl, gr, eap2, vap2, wev[None], attv[None])
        nump, denp = _k4(wrows, exc.reshape(EP), dstp, dst3, zr, zn)
        den = _k5a(denp.reshape(NW, NB, BN)).reshape(N, 1)
        return nump.reshape(NC, NP, F), den

    nump1, den1 = layer(xl1, xr1, We1[:, 0], att1)
    xl2, xr2 = _k5(nump1, den1, b1[None], Wl2.T, Wr2.T, bl2[None], br2[None])
    nump2, den2 = layer(xl2, xr2, We2[:, 0], att2)
    return _k5f(nump2, den2, b2[None])


# speedup vs baseline: 1.0806x; 1.0806x over previous
import functools

import jax
import jax.numpy as jnp
from jax import lax
from jax.experimental import pallas as pl
from jax.experimental.pallas import tpu as pltpu
from jax.experimental.pallas import tpu_sc as plsc

N = 10000
E = 320000
F = 128

NC = 2
NS = 16
NW = NC * NS
L = 16

EP = 335872
CH = 128
EW = EP // NW
CPW = EW // CH

EPW0 = E // NW
CH0 = 80
CPW0 = EPW0 // CH0

NP = 10240
RPT = NP // NS

NB = 10
BN = N // NB
BE = 2048
GE = EP // BE

def _sc_preproc_body(src_hbm, dst_hbm, ew_hbm, zn_hbm,
                     ea0_hbm, valid_hbm, cnt_hbm, eas_hbm,
                     sall, dall, wall, eaall, vall, cnt_v, eas_v):
    c = lax.axis_index("c")
    s = lax.axis_index("s")
    w = c * NS + s
    b0 = w * EPW0
    pltpu.sync_copy(zn_hbm, cnt_v)
    pltpu.sync_copy(zn_hbm, eas_v)
    pltpu.sync_copy(src_hbm.at[pl.ds(b0, EPW0)], sall)
    pltpu.sync_copy(dst_hbm.at[pl.ds(b0, EPW0)], dall)
    pltpu.sync_copy(ew_hbm.at[pl.ds(b0, EPW0)], wall)

    UNROLL = 5

    def step(i, carry):
        for j in range(UNROLL):
            sl = pl.ds((i * UNROLL + j) * L, L)
            sv = sall[sl]
            dv = dall[sl]
            wv = wall[sl]
            mk = jnp.where(sv != dv, jnp.float32(1.0), jnp.float32(0.0))
            ea = mk * wv
            eaall[sl] = ea
            vall[sl] = mk
            plsc.addupdate_scatter(cnt_v, [dv], mk)
            plsc.addupdate_scatter(eas_v, [dv], ea)
        return carry

    lax.fori_loop(0, EPW0 // (L * UNROLL), step, 0)
    pltpu.sync_copy(eaall, ea0_hbm.at[pl.ds(b0, EPW0)])
    pltpu.sync_copy(vall, valid_hbm.at[pl.ds(b0, EPW0)])
    pltpu.sync_copy(cnt_v, cnt_hbm.at[pl.ds(w * N, N)])
    pltpu.sync_copy(eas_v, eas_hbm.at[pl.ds(w * N, N)])


@functools.lru_cache(maxsize=None)
def _sc_kernels():
    mesh = plsc.VectorSubcoreMesh(
        core_axis_name="c", subcore_axis_name="s",
        num_cores=NC, num_subcores=NS,
    )
    cparams = pltpu.CompilerParams(needs_layout_passes=False)
    k0 = pl.kernel(
        _sc_preproc_body,
        out_type=[
            jax.ShapeDtypeStruct((E,), jnp.float32),
            jax.ShapeDtypeStruct((E,), jnp.float32),
            jax.ShapeDtypeStruct((NW * N,), jnp.float32),
            jax.ShapeDtypeStruct((NW * N,), jnp.float32),
        ],
        mesh=mesh,
        compiler_params=cparams,
        scratch_types=[
            pltpu.VMEM((EPW0,), jnp.int32),
            pltpu.VMEM((EPW0,), jnp.int32),
            pltpu.VMEM((EPW0,), jnp.float32),
            pltpu.VMEM((EPW0,), jnp.float32),
            pltpu.VMEM((EPW0,), jnp.float32),
            pltpu.VMEM((N,), jnp.float32),
            pltpu.VMEM((N,), jnp.float32),
        ],
    )
    k2 = pl.kernel(
        _sc_gather_body,
        out_type=[
            jax.ShapeDtypeStruct((EP, F), jnp.float32),
            jax.ShapeDtypeStruct((EP, F), jnp.float32),
        ],
        mesh=mesh,
        compiler_params=cparams,
        scratch_types=[
            pltpu.VMEM((CH,), jnp.int32),
            pltpu.VMEM((CH,), jnp.int32),
            pltpu.VMEM((CH,), jnp.int32),
            pltpu.VMEM((CH,), jnp.int32),
            pltpu.VMEM((CH, F), jnp.float32),
            pltpu.VMEM((CH, F), jnp.float32),
            pltpu.VMEM((CH, F), jnp.float32),
            pltpu.VMEM((CH, F), jnp.float32),
            pltpu.SemaphoreType.DMA,
            pltpu.SemaphoreType.DMA,
            pltpu.SemaphoreType.DMA,
            pltpu.SemaphoreType.DMA,
            pltpu.SemaphoreType.DMA,
        ],
    )
    k4 = pl.kernel(
        _sc_scatter_body,
        out_type=[
            jax.ShapeDtypeStruct((NC * NP, F), jnp.float32),
            jax.ShapeDtypeStruct((NW * N,), jnp.float32),
        ],
        mesh=mesh,
        compiler_params=cparams,
        scratch_types=[
            pltpu.VMEM((2, CH), jnp.int32),
            pltpu.VMEM((2 * CH,), jnp.int32),
            pltpu.VMEM((2 * CH,), jnp.float32),
            pltpu.VMEM((CH, F), jnp.float32),
            pltpu.VMEM((CH, F), jnp.float32),
            pltpu.VMEM((N,), jnp.float32),
            pltpu.VMEM_SHARED((NP, F), jnp.float32),
            pltpu.SemaphoreType.DMA,
            pltpu.SemaphoreType.DMA,
        ],
    )
    kf = pl.kernel(
        _sc_layer_body,
        out_type=[
            jax.ShapeDtypeStruct((NC * NP, F), jnp.float32),
            jax.ShapeDtypeStruct((NW * N,), jnp.float32),
        ],
        mesh=mesh,
        compiler_params=cparams,
        scratch_types=[
            pltpu.VMEM((DEP, CHF), jnp.int32),
            pltpu.VMEM((DEP, CHF), jnp.int32),
            pltpu.VMEM((DEP * CHF,), jnp.int32),
            pltpu.VMEM((DEP * CHF,), jnp.float32),
            pltpu.VMEM((DEP * CHF,), jnp.float32),
            pltpu.VMEM((DEP * CHF,), jnp.float32),
            pltpu.VMEM((F,), jnp.float32),
            pltpu.VMEM((F,), jnp.float32),
            pltpu.VMEM((L,), jnp.float32),
            pltpu.VMEM((CHF, F), jnp.float32),
            pltpu.VMEM((CHF, F), jnp.float32),
            pltpu.VMEM((CHF, F), jnp.float32),
            pltpu.VMEM((CHF, F), jnp.float32),
            pltpu.VMEM((N,), jnp.float32),
            pltpu.VMEM_SHARED((NP, F), jnp.float32),
            pltpu.SemaphoreType.DMA,
            pltpu.SemaphoreType.DMA,
            pltpu.SemaphoreType.DMA,
            pltpu.SemaphoreType.DMA,
        ],
    )
    return k0, k2, k4, kf


DEP = 2


def _sc_gather_body(xl_hbm, xr_hbm, src_hbm, dst_hbm, gl_hbm, gr_hbm,
                    si0, di0, si1, di1, bl0, br0, bl1, br1,
                    is0, is1, gs0, gs1, wsem):
    c = lax.axis_index("c")
    s = lax.axis_index("s")
    w = c * NS + s
    b0 = w * EW
    bufs = ((si0, di0, bl0, br0, is0, gs0), (si1, di1, bl1, br1, is1, gs1))

    def it(t, carry):
        i0 = t * DEP
        icps = []
        for k in range(DEP):
            sik, dik, blk, brk, isk, gsk = bufs[k]
            base = b0 + (i0 + k) * CH
            icps.append((
                pltpu.async_copy(src_hbm.at[pl.ds(base, CH)], sik, isk),
                pltpu.async_copy(dst_hbm.at[pl.ds(base, CH)], dik, isk),
            ))
        gcps = []
        for k in range(DEP):
            sik, dik, blk, brk, isk, gsk = bufs[k]
            icps[k][0].wait()
            icps[k][1].wait()
            gcps.append((
                pltpu.async_copy(xl_hbm.at[sik], blk, gsk),
                pltpu.async_copy(xr_hbm.at[dik], brk, gsk),
            ))
        wcps = []
        for k in range(DEP):
            gcps[k][0].wait()
            gcps[k][1].wait()
            base = b0 + (i0 + k) * CH
            wcps.append(pltpu.async_copy(bufs[k][2], gl_hbm.at[pl.ds(base, CH)], wsem))
            wcps.append(pltpu.async_copy(bufs[k][3], gr_hbm.at[pl.ds(base, CH)], wsem))
        for cp in wcps:
            cp.wait()
        return carry

    lax.fori_loop(0, CPW // DEP, it, 0)


def _sc_scatter_body(wr_hbm, ex_hbm, dst_hbm, dst3_hbm, zr_hbm, zn_hbm,
                     nump_hbm, denp_hbm,
                     di2, di1, exb, bw0, bw1, den_v, num_s,
                     ls0, ls1):
    c = lax.axis_index("c")
    s = lax.axis_index("s")
    w = c * NS + s
    r0 = s * RPT
    b0 = w * EW
    pltpu.sync_copy(zn_hbm, den_v)
    pltpu.sync_copy(zr_hbm.at[pl.ds(r0, RPT)], num_s.at[pl.ds(r0, RPT)])
    plsc.subcore_barrier()
    bufs = ((bw0, ls0), (bw1, ls1))

    def it(t, carry):
        i0 = t * DEP
        lcps = []
        for k in range(DEP):
            bwk, sk = bufs[k]
            i = i0 + k
            base = b0 + i * CH
            lcps.append((
                pltpu.async_copy(wr_hbm.at[pl.ds(base, CH)], bwk, sk),
                pltpu.async_copy(dst3_hbm.at[w, i], di2.at[k], sk),
                pltpu.async_copy(dst_hbm.at[pl.ds(base, CH)],
                                 di1.at[pl.ds(k * CH, CH)], sk),
                pltpu.async_copy(ex_hbm.at[pl.ds(base, CH)],
                                 exb.at[pl.ds(k * CH, CH)], sk),
            ))
        for k in range(DEP):
            for cp in lcps[k]:
                cp.wait()
            pltpu.sync_copy(bufs[k][0], num_s.at[di2.at[k]], add=True)
            for j in range(CH // L):
                sl = pl.ds(k * CH + j * L, L)
                plsc.addupdate_scatter(den_v, [di1[sl]], exb[sl])
        return carry

    lax.fori_loop(0, CPW // DEP, it, 0)
    plsc.subcore_barrier()
    pltpu.sync_copy(den_v, denp_hbm.at[pl.ds(w * N, N)])
    pltpu.sync_copy(num_s.at[pl.ds(r0, RPT)],
                    nump_hbm.at[pl.ds(c * NP + r0, RPT)])


CHF = 64
CPWF = EW // CHF


def _sc_layer_body(xl_hbm, xr_hbm, src3_hbm, dst3_hbm, dst_hbm, ea_hbm,
                   va_hbm, we_hbm, att_hbm, zr_hbm, zn_hbm,
                   nump_hbm, denp_hbm,
                   si2, di2, di1, eab, vab, exb, wev, attv, tmp,
                   bl0, br0, bl1, br1, den_v, num_s,
                   ls0, ls1, gs0, gs1):
    c = lax.axis_index("c")
    s = lax.axis_index("s")
    w = c * NS + s
    r0 = s * RPT
    b0 = w * EW
    pltpu.sync_copy(zn_hbm, den_v)
    pltpu.sync_copy(zr_hbm.at[pl.ds(r0, RPT)], num_s.at[pl.ds(r0, RPT)])
    pltpu.sync_copy(we_hbm, wev)
    pltpu.sync_copy(att_hbm, attv)
    plsc.subcore_barrier()
    lanes = lax.broadcasted_iota(jnp.int32, (L,), 0)
    bufs = ((bl0, br0, ls0, gs0), (bl1, br1, ls1, gs1))

    def it(t, carry):
        i0 = t * DEP
        lcps = []
        for k in range(DEP):
            blk, brk, lsk, gsk = bufs[k]
            i = i0 + k
            base = b0 + i * CHF
            lcps.append((
                pltpu.async_copy(src3_hbm.at[w, i], si2.at[k], lsk),
                pltpu.async_copy(dst3_hbm.at[w, i], di2.at[k], lsk),
                pltpu.async_copy(dst_hbm.at[pl.ds(base, CHF)],
                                 di1.at[pl.ds(k * CHF, CHF)], lsk),
                pltpu.async_copy(ea_hbm.at[pl.ds(base, CHF)],
                                 eab.at[pl.ds(k * CHF, CHF)], lsk),
                pltpu.async_copy(va_hbm.at[pl.ds(base, CHF)],
                                 vab.at[pl.ds(k * CHF, CHF)], lsk),
            ))
        gcps = []
        for k in range(DEP):
            blk, brk, lsk, gsk = bufs[k]
            for cp in lcps[k]:
                cp.wait()
            gcps.append((
                pltpu.async_copy(xl_hbm.at[si2.at[k]], blk, gsk),
                pltpu.async_copy(xr_hbm.at[di2.at[k]], brk, gsk),
            ))
        for k in range(DEP):
            blk, brk, lsk, gsk = bufs[k]
            gcps[k][0].wait()
            gcps[k][1].wait()

            def edge(e, carry2):
                easp = plsc.load_gather(eab, [jnp.full((L,), k * CHF, jnp.int32) + e])
                acc = jnp.zeros((L,), jnp.float32)
                for cslice in range(F // L):
                    sl = pl.ds(cslice * L, L)
                    m = blk[e, sl] + brk[e, sl] + easp * wev[sl]
                    m = jnp.maximum(m, 0.2 * m)
                    acc = acc + attv[sl] * m
                tmp[:] = plsc.cumsum(acc)
                esp = plsc.load_gather(tmp, [jnp.full((L,), L - 1, jnp.int32)])
                vasp = plsc.load_gather(vab, [jnp.full((L,), k * CHF, jnp.int32) + e])
                ex = vasp * jnp.exp(esp)
                plsc.store_scatter(exb, [jnp.full((L,), k * CHF, jnp.int32) + e], ex)
                for cslice in range(F // L):
                    sl = pl.ds(cslice * L, L)
                    blk[e, sl] = blk[e, sl] * ex
                return carry2

            lax.fori_loop(0, CHF, edge, 0)
            pltpu.sync_copy(blk, num_s.at[di2.at[k]], add=True)
            for j in range(CHF // L):
                sl = pl.ds(k * CHF + j * L, L)
                plsc.addupdate_scatter(den_v, [di1[sl]], exb[sl])
        return carry

    lax.fori_loop(0, CPWF // DEP, it, 0)
    plsc.subcore_barrier()
    pltpu.sync_copy(den_v, denp_hbm.at[pl.ds(w * N, N)])
    pltpu.sync_copy(num_s.at[pl.ds(r0, RPT)],
                    nump_hbm.at[pl.ds(c * NP + r0, RPT)])


def _dot(a, b):
    return jnp.dot(a, b, preferred_element_type=jnp.float32,
                   precision=lax.Precision.HIGHEST)


def _tc_prep_body(x_ref, wlt_ref, wrt_ref, bl_ref, br_ref, cnt_ref, eas_ref,
                  xl_ref, xr_ref, la_ref):
    xb = x_ref[...]
    xl_ref[...] = _dot(xb, wlt_ref[...]) + bl_ref[...]
    xr_ref[...] = _dot(xb, wrt_ref[...]) + br_ref[...]

    @pl.when(pl.program_id(0) == 0)
    def _():
        cnt = jnp.sum(cnt_ref[...], axis=0)
        eas = jnp.sum(eas_ref[...], axis=0)
        la_ref[...] = eas / jnp.maximum(cnt, 1.0)


_k1 = pl.pallas_call(
    _tc_prep_body,
    grid=(NB,),
    in_specs=[
        pl.BlockSpec((BN, F), lambda i: (i, 0)),
        pl.BlockSpec((F, F), lambda i: (0, 0)),
        pl.BlockSpec((F, F), lambda i: (0, 0)),
        pl.BlockSpec((1, F), lambda i: (0, 0)),
        pl.BlockSpec((1, F), lambda i: (0, 0)),
        pl.BlockSpec((NW, NB, BN), lambda i: (0, 0, 0)),
        pl.BlockSpec((NW, NB, BN), lambda i: (0, 0, 0)),
    ],
    out_specs=[
        pl.BlockSpec((BN, F), lambda i: (i, 0)),
        pl.BlockSpec((BN, F), lambda i: (i, 0)),
        pl.BlockSpec((NB, BN), lambda i: (0, 0)),
    ],
    out_shape=[
        jax.ShapeDtypeStruct((N, F), jnp.float32),
        jax.ShapeDtypeStruct((N, F), jnp.float32),
        jax.ShapeDtypeStruct((NB, BN), jnp.float32),
    ],
)


def _tc_edge_body(gl_ref, gr_ref, ea_ref, va_ref, we_ref, att_ref,
                  wr_ref, ex_ref):
    gl = gl_ref[...]
    m = gl + gr_ref[...] + ea_ref[...] * we_ref[...]
    m = jnp.maximum(m, 0.2 * m)
    e = jnp.sum(m * att_ref[...], axis=1, keepdims=True)
    ex = va_ref[...] * jnp.exp(e)
    ex_ref[...] = ex
    wr_ref[...] = gl * ex


_k3 = pl.pallas_call(
    _tc_edge_body,
    grid=(GE,),
    in_specs=[
        pl.BlockSpec((BE, F), lambda i: (i, 0)),
        pl.BlockSpec((BE, F), lambda i: (i, 0)),
        pl.BlockSpec((BE, 1), lambda i: (i, 0)),
        pl.BlockSpec((BE, 1), lambda i: (i, 0)),
        pl.BlockSpec((1, F), lambda i: (0, 0)),
        pl.BlockSpec((1, F), lambda i: (0, 0)),
    ],
    out_specs=[
        pl.BlockSpec((BE, F), lambda i: (i, 0)),
        pl.BlockSpec((BE, 1), lambda i: (i, 0)),
    ],
    out_shape=[
        jax.ShapeDtypeStruct((EP, F), jnp.float32),
        jax.ShapeDtypeStruct((EP, 1), jnp.float32),
    ],
)


def _tc_den_body(dp_ref, out_ref):
    out_ref[...] = jnp.sum(dp_ref[...], axis=0)


_k5a = pl.pallas_call(
    _tc_den_body,
    out_shape=jax.ShapeDtypeStruct((NB, BN), jnp.float32),
)


def _tc_comb_body(np_ref, den_ref, b_ref, wlt_ref, wrt_ref, bl_ref, br_ref,
                  xl_ref, xr_ref):
    ns = np_ref[0] + np_ref[1]
    h = ns / den_ref[...] + b_ref[...]
    h = jnp.maximum(h, 0.0)
    xl_ref[...] = _dot(h, wlt_ref[...]) + bl_ref[...]
    xr_ref[...] = _dot(h, wrt_ref[...]) + br_ref[...]


_k5 = pl.pallas_call(
    _tc_comb_body,
    grid=(NB,),
    in_specs=[
        pl.BlockSpec((NC, BN, F), lambda i: (0, i, 0)),
        pl.BlockSpec((BN, 1), lambda i: (i, 0)),
        pl.BlockSpec((1, F), lambda i: (0, 0)),
        pl.BlockSpec((F, F), lambda i: (0, 0)),
        pl.BlockSpec((F, F), lambda i: (0, 0)),
        pl.BlockSpec((1, F), lambda i: (0, 0)),
        pl.BlockSpec((1, F), lambda i: (0, 0)),
    ],
    out_specs=[
        pl.BlockSpec((BN, F), lambda i: (i, 0)),
        pl.BlockSpec((BN, F), lambda i: (i, 0)),
    ],
    out_shape=[
        jax.ShapeDtypeStruct((N, F), jnp.float32),
        jax.ShapeDtypeStruct((N, F), jnp.float32),
    ],
)


def _tc_final_body(np_ref, den_ref, b_ref, out_ref):
    out_ref[...] = (np_ref[0] + np_ref[1]) / den_ref[...] + b_ref[...]


_k5f = pl.pallas_call(
    _tc_final_body,
    grid=(NB,),
    in_specs=[
        pl.BlockSpec((NC, BN, F), lambda i: (0, i, 0)),
        pl.BlockSpec((BN, 1), lambda i: (i, 0)),
        pl.BlockSpec((1, F), lambda i: (0, 0)),
    ],
    out_specs=pl.BlockSpec((BN, F), lambda i: (i, 0)),
    out_shape=jax.ShapeDtypeStruct((N, F), jnp.float32),
)


def kernel(x, edge_index, edge_weight, Wl1, bl1, Wr1, br1, We1, att1, b1,
           Wl2, bl2, Wr2, br2, We2, att2, b2):
    src0 = edge_index[0]
    dst0 = edge_index[1]
    ew = edge_weight[:, 0]
    zn = jnp.zeros((N,), jnp.float32)
    zr = jnp.zeros((NP, F), jnp.float32)

    _k0, _k2, _k4, _kf = _sc_kernels()
    ea0, valid0, cnt_p, eas_p = _k0(src0, dst0, ew, zn)
    xl1, xr1, la = _k1(x, Wl1.T, Wr1.T, bl1[None], br1[None],
                       cnt_p.reshape(NW, NB, BN), eas_p.reshape(NW, NB, BN))

    loop_idx = jnp.arange(N, dtype=jnp.int32)
    pad = EP - E - N
    pad_idx = jnp.arange(pad, dtype=jnp.int32) % N
    srcp = jnp.concatenate([src0, loop_idx, pad_idx])
    dstp = jnp.concatenate([dst0, loop_idx, pad_idx])
    eap = jnp.concatenate([ea0, la.reshape(N), jnp.zeros((pad,), jnp.float32)])
    vap = jnp.concatenate([valid0, jnp.ones((N,), jnp.float32),
                           jnp.zeros((pad,), jnp.float32)])
    eap2 = eap[:, None]
    vap2 = vap[:, None]

    src3f = srcp.reshape(NW, CPWF, CHF)
    dst3f = dstp.reshape(NW, CPWF, CHF)

    def layer(xl, xr, wev, attv):
        nump, denp = _kf(xl, xr, src3f, dst3f, dstp, eap, vap,
                         wev, attv, zr, zn)
        den = _k5a(denp.reshape(NW, NB, BN)).reshape(N, 1)
        return nump.reshape(NC, NP, F), den

    nump1, den1 = layer(xl1, xr1, We1[:, 0], att1)
    xl2, xr2 = _k5(nump1, den1, b1[None], Wl2.T, Wr2.T, bl2[None], br2[None])
    nump2, den2 = layer(xl2, xr2, We2[:, 0], att2)
    return _k5f(nump2, den2, b2[None])


# speedup vs baseline: 1.2438x; 1.1510x over previous
import functools

import jax
import jax.numpy as jnp
from jax import lax
from jax.experimental import pallas as pl
from jax.experimental.pallas import tpu as pltpu
from jax.experimental.pallas import tpu_sc as plsc

N = 10000
E = 320000
F = 128

NC = 2
NS = 16
NW = NC * NS
L = 16

EP = 335872
CH = 128
EW = EP // NW
CPW = EW // CH

EPW0 = E // NW
CH0 = 80
CPW0 = EPW0 // CH0

NP = 10240
RPT = NP // NS

NB = 10
BN = N // NB
BE = 2048
GE = EP // BE

def _sc_preproc_body(src_hbm, dst_hbm, ew_hbm, zn_hbm,
                     ea0_hbm, valid_hbm, cnt_hbm, eas_hbm,
                     sall, dall, wall, eaall, vall, cnt_v, eas_v):
    c = lax.axis_index("c")
    s = lax.axis_index("s")
    w = c * NS + s
    b0 = w * EPW0
    pltpu.sync_copy(zn_hbm, cnt_v)
    pltpu.sync_copy(zn_hbm, eas_v)
    pltpu.sync_copy(src_hbm.at[pl.ds(b0, EPW0)], sall)
    pltpu.sync_copy(dst_hbm.at[pl.ds(b0, EPW0)], dall)
    pltpu.sync_copy(ew_hbm.at[pl.ds(b0, EPW0)], wall)

    UNROLL = 5

    def step(i, carry):
        for j in range(UNROLL):
            sl = pl.ds((i * UNROLL + j) * L, L)
            sv = sall[sl]
            dv = dall[sl]
            wv = wall[sl]
            mk = jnp.where(sv != dv, jnp.float32(1.0), jnp.float32(0.0))
            ea = mk * wv
            eaall[sl] = ea
            vall[sl] = mk
            plsc.addupdate_scatter(cnt_v, [dv], mk)
            plsc.addupdate_scatter(eas_v, [dv], ea)
        return carry

    lax.fori_loop(0, EPW0 // (L * UNROLL), step, 0)
    pltpu.sync_copy(eaall, ea0_hbm.at[pl.ds(b0, EPW0)])
    pltpu.sync_copy(vall, valid_hbm.at[pl.ds(b0, EPW0)])
    pltpu.sync_copy(cnt_v, cnt_hbm.at[pl.ds(w * N, N)])
    pltpu.sync_copy(eas_v, eas_hbm.at[pl.ds(w * N, N)])


@functools.lru_cache(maxsize=None)
def _sc_kernels():
    mesh = plsc.VectorSubcoreMesh(
        core_axis_name="c", subcore_axis_name="s",
        num_cores=NC, num_subcores=NS,
    )
    cparams = pltpu.CompilerParams(needs_layout_passes=False)
    k0 = pl.kernel(
        _sc_preproc_body,
        out_type=[
            jax.ShapeDtypeStruct((E,), jnp.float32),
            jax.ShapeDtypeStruct((E,), jnp.float32),
            jax.ShapeDtypeStruct((NW * N,), jnp.float32),
            jax.ShapeDtypeStruct((NW * N,), jnp.float32),
        ],
        mesh=mesh,
        compiler_params=cparams,
        scratch_types=[
            pltpu.VMEM((EPW0,), jnp.int32),
            pltpu.VMEM((EPW0,), jnp.int32),
            pltpu.VMEM((EPW0,), jnp.float32),
            pltpu.VMEM((EPW0,), jnp.float32),
            pltpu.VMEM((EPW0,), jnp.float32),
            pltpu.VMEM((N,), jnp.float32),
            pltpu.VMEM((N,), jnp.float32),
        ],
    )
    k2 = pl.kernel(
        _sc_gather_body,
        out_type=[
            jax.ShapeDtypeStruct((EP, F), jnp.float32),
            jax.ShapeDtypeStruct((EP, F), jnp.float32),
        ],
        mesh=mesh,
        compiler_params=cparams,
        scratch_types=[
            pltpu.VMEM((CH,), jnp.int32),
            pltpu.VMEM((CH,), jnp.int32),
            pltpu.VMEM((CH,), jnp.int32),
            pltpu.VMEM((CH,), jnp.int32),
            pltpu.VMEM((CH, F), jnp.float32),
            pltpu.VMEM((CH, F), jnp.float32),
            pltpu.VMEM((CH, F), jnp.float32),
            pltpu.VMEM((CH, F), jnp.float32),
            pltpu.SemaphoreType.DMA,
            pltpu.SemaphoreType.DMA,
            pltpu.SemaphoreType.DMA,
            pltpu.SemaphoreType.DMA,
            pltpu.SemaphoreType.DMA,
        ],
    )
    k4 = pl.kernel(
        _sc_scatter_body,
        out_type=[
            jax.ShapeDtypeStruct((NC * NP, F), jnp.float32),
            jax.ShapeDtypeStruct((NW * N,), jnp.float32),
        ],
        mesh=mesh,
        compiler_params=cparams,
        scratch_types=[
            pltpu.VMEM((2, CH), jnp.int32),
            pltpu.VMEM((2 * CH,), jnp.int32),
            pltpu.VMEM((2 * CH,), jnp.float32),
            pltpu.VMEM((CH, F), jnp.float32),
            pltpu.VMEM((CH, F), jnp.float32),
            pltpu.VMEM((N,), jnp.float32),
            pltpu.VMEM_SHARED((NP, F), jnp.float32),
            pltpu.SemaphoreType.DMA,
            pltpu.SemaphoreType.DMA,
        ],
    )
    kf = pl.kernel(
        _sc_layer_body,
        out_type=[
            jax.ShapeDtypeStruct((NC * NP, F), jnp.float32),
            jax.ShapeDtypeStruct((NW * N,), jnp.float32),
        ],
        mesh=mesh,
        compiler_params=cparams,
        scratch_types=[
            pltpu.VMEM((DEP, CHF), jnp.int32),
            pltpu.VMEM((DEP, CHF), jnp.int32),
            pltpu.VMEM((DEP * CHF,), jnp.int32),
            pltpu.VMEM((DEP * CHF,), jnp.float32),
            pltpu.VMEM((DEP * CHF,), jnp.float32),
            pltpu.VMEM((DEP * CHF,), jnp.float32),
            pltpu.VMEM((F,), jnp.float32),
            pltpu.VMEM((F,), jnp.float32),
            pltpu.VMEM((2 * L,), jnp.float32),
            pltpu.VMEM((CHF, F), jnp.float32),
            pltpu.VMEM((CHF, F), jnp.float32),
            pltpu.VMEM((CHF, F), jnp.float32),
            pltpu.VMEM((CHF, F), jnp.float32),
            pltpu.VMEM((N,), jnp.float32),
            pltpu.VMEM_SHARED((NP, F), jnp.float32),
            pltpu.SemaphoreType.DMA,
            pltpu.SemaphoreType.DMA,
            pltpu.SemaphoreType.DMA,
            pltpu.SemaphoreType.DMA,
        ],
    )
    return k0, k2, k4, kf


DEP = 2


def _sc_gather_body(xl_hbm, xr_hbm, src_hbm, dst_hbm, gl_hbm, gr_hbm,
                    si0, di0, si1, di1, bl0, br0, bl1, br1,
                    is0, is1, gs0, gs1, wsem):
    c = lax.axis_index("c")
    s = lax.axis_index("s")
    w = c * NS + s
    b0 = w * EW
    bufs = ((si0, di0, bl0, br0, is0, gs0), (si1, di1, bl1, br1, is1, gs1))

    def it(t, carry):
        i0 = t * DEP
        icps = []
        for k in range(DEP):
            sik, dik, blk, brk, isk, gsk = bufs[k]
            base = b0 + (i0 + k) * CH
            icps.append((
                pltpu.async_copy(src_hbm.at[pl.ds(base, CH)], sik, isk),
                pltpu.async_copy(dst_hbm.at[pl.ds(base, CH)], dik, isk),
            ))
        gcps = []
        for k in range(DEP):
            sik, dik, blk, brk, isk, gsk = bufs[k]
            icps[k][0].wait()
            icps[k][1].wait()
            gcps.append((
                pltpu.async_copy(xl_hbm.at[sik], blk, gsk),
                pltpu.async_copy(xr_hbm.at[dik], brk, gsk),
            ))
        wcps = []
        for k in range(DEP):
            gcps[k][0].wait()
            gcps[k][1].wait()
            base = b0 + (i0 + k) * CH
            wcps.append(pltpu.async_copy(bufs[k][2], gl_hbm.at[pl.ds(base, CH)], wsem))
            wcps.append(pltpu.async_copy(bufs[k][3], gr_hbm.at[pl.ds(base, CH)], wsem))
        for cp in wcps:
            cp.wait()
        return carry

    lax.fori_loop(0, CPW // DEP, it, 0)


def _sc_scatter_body(wr_hbm, ex_hbm, dst_hbm, dst3_hbm, zr_hbm, zn_hbm,
                     nump_hbm, denp_hbm,
                     di2, di1, exb, bw0, bw1, den_v, num_s,
                     ls0, ls1):
    c = lax.axis_index("c")
    s = lax.axis_index("s")
    w = c * NS + s
    r0 = s * RPT
    b0 = w * EW
    pltpu.sync_copy(zn_hbm, den_v)
    pltpu.sync_copy(zr_hbm.at[pl.ds(r0, RPT)], num_s.at[pl.ds(r0, RPT)])
    plsc.subcore_barrier()
    bufs = ((bw0, ls0), (bw1, ls1))

    def it(t, carry):
        i0 = t * DEP
        lcps = []
        for k in range(DEP):
            bwk, sk = bufs[k]
            i = i0 + k
            base = b0 + i * CH
            lcps.append((
                pltpu.async_copy(wr_hbm.at[pl.ds(base, CH)], bwk, sk),
                pltpu.async_copy(dst3_hbm.at[w, i], di2.at[k], sk),
                pltpu.async_copy(dst_hbm.at[pl.ds(base, CH)],
                                 di1.at[pl.ds(k * CH, CH)], sk),
                pltpu.async_copy(ex_hbm.at[pl.ds(base, CH)],
                                 exb.at[pl.ds(k * CH, CH)], sk),
            ))
        for k in range(DEP):
            for cp in lcps[k]:
                cp.wait()
            pltpu.sync_copy(bufs[k][0], num_s.at[di2.at[k]], add=True)
            for j in range(CH // L):
                sl = pl.ds(k * CH + j * L, L)
                plsc.addupdate_scatter(den_v, [di1[sl]], exb[sl])
        return carry

    lax.fori_loop(0, CPW // DEP, it, 0)
    plsc.subcore_barrier()
    pltpu.sync_copy(den_v, denp_hbm.at[pl.ds(w * N, N)])
    pltpu.sync_copy(num_s.at[pl.ds(r0, RPT)],
                    nump_hbm.at[pl.ds(c * NP + r0, RPT)])


CHF = 64
CPWF = EW // CHF


def _sc_layer_body(xl_hbm, xr_hbm, src3_hbm, dst3_hbm, dst_hbm, ea_hbm,
                   va_hbm, we_hbm, att_hbm, zr_hbm, zn_hbm,
                   nump_hbm, denp_hbm,
                   si2, di2, di1, eab, vab, exb, wev, attv, tmp,
                   bl0, br0, bl1, br1, den_v, num_s,
                   ls0, ls1, gs0, gs1):
    c = lax.axis_index("c")
    s = lax.axis_index("s")
    w = c * NS + s
    r0 = s * RPT
    b0 = w * EW
    pltpu.sync_copy(zn_hbm, den_v)
    pltpu.sync_copy(zr_hbm.at[pl.ds(r0, RPT)], num_s.at[pl.ds(r0, RPT)])
    pltpu.sync_copy(we_hbm, wev)
    pltpu.sync_copy(att_hbm, attv)
    plsc.subcore_barrier()
    lanes = lax.broadcasted_iota(jnp.int32, (L,), 0)
    bufs = ((bl0, br0, ls0, gs0), (bl1, br1, ls1, gs1))

    def it(t, carry):
        i0 = t * DEP
        lcps = []
        for k in range(DEP):
            blk, brk, lsk, gsk = bufs[k]
            i = i0 + k
            base = b0 + i * CHF
            lcps.append((
                pltpu.async_copy(src3_hbm.at[w, i], si2.at[k], lsk),
                pltpu.async_copy(dst3_hbm.at[w, i], di2.at[k], lsk),
                pltpu.async_copy(dst_hbm.at[pl.ds(base, CHF)],
                                 di1.at[pl.ds(k * CHF, CHF)], lsk),
                pltpu.async_copy(ea_hbm.at[pl.ds(base, CHF)],
                                 eab.at[pl.ds(k * CHF, CHF)], lsk),
                pltpu.async_copy(va_hbm.at[pl.ds(base, CHF)],
                                 vab.at[pl.ds(k * CHF, CHF)], lsk),
            ))
        gcps = []
        for k in range(DEP):
            blk, brk, lsk, gsk = bufs[k]
            for cp in lcps[k]:
                cp.wait()
            gcps.append((
                pltpu.async_copy(xl_hbm.at[si2.at[k]], blk, gsk),
                pltpu.async_copy(xr_hbm.at[di2.at[k]], brk, gsk),
            ))
        for k in range(DEP):
            blk, brk, lsk, gsk = bufs[k]
            gcps[k][0].wait()
            gcps[k][1].wait()

            kofs = jnp.full((L,), k * CHF, jnp.int32)
            l15 = jnp.full((L,), L - 1, jnp.int32)

            def edge(e2, carry2):
                evs = []
                for u in range(2):
                    e = e2 * 2 + u
                    easp = plsc.load_gather(eab, [kofs + e])
                    acc = jnp.zeros((L,), jnp.float32)
                    vals = []
                    for cslice in range(F // L):
                        sl = pl.ds(cslice * L, L)
                        bv = blk[e, sl]
                        vals.append(bv)
                        m = bv + brk[e, sl] + easp * wev[sl]
                        m = jnp.maximum(m, 0.2 * m)
                        acc = acc + attv[sl] * m
                    evs.append((e, acc, vals))
                tmp[pl.ds(0, L)] = plsc.cumsum(evs[0][1])
                tmp[pl.ds(L, L)] = plsc.cumsum(evs[1][1])
                for u in range(2):
                    e, acc, vals = evs[u]
                    esp = plsc.load_gather(tmp, [l15 + u * L])
                    vasp = plsc.load_gather(vab, [kofs + e])
                    ex = vasp * jnp.exp(esp)
                    plsc.store_scatter(exb, [kofs + e], ex)
                    for cslice in range(F // L):
                        sl = pl.ds(cslice * L, L)
                        blk[e, sl] = vals[cslice] * ex
                return carry2

            lax.fori_loop(0, CHF // 2, edge, 0)
            pltpu.sync_copy(blk, num_s.at[di2.at[k]], add=True)
            for j in range(CHF // L):
                sl = pl.ds(k * CHF + j * L, L)
                plsc.addupdate_scatter(den_v, [di1[sl]], exb[sl])
        return carry

    lax.fori_loop(0, CPWF // DEP, it, 0)
    plsc.subcore_barrier()
    pltpu.sync_copy(den_v, denp_hbm.at[pl.ds(w * N, N)])
    pltpu.sync_copy(num_s.at[pl.ds(r0, RPT)],
                    nump_hbm.at[pl.ds(c * NP + r0, RPT)])


def _dot(a, b):
    return jnp.dot(a, b, preferred_element_type=jnp.float32,
                   precision=lax.Precision.HIGHEST)


def _tc_prep_body(x_ref, wlt_ref, wrt_ref, bl_ref, br_ref, cnt_ref, eas_ref,
                  xl_ref, xr_ref, la_ref):
    xb = x_ref[...]
    xl_ref[...] = _dot(xb, wlt_ref[...]) + bl_ref[...]
    xr_ref[...] = _dot(xb, wrt_ref[...]) + br_ref[...]

    @pl.when(pl.program_id(0) == 0)
    def _():
        cnt = jnp.sum(cnt_ref[...], axis=0)
        eas = jnp.sum(eas_ref[...], axis=0)
        la_ref[...] = eas / jnp.maximum(cnt, 1.0)


_k1 = pl.pallas_call(
    _tc_prep_body,
    grid=(NB,),
    in_specs=[
        pl.BlockSpec((BN, F), lambda i: (i, 0)),
        pl.BlockSpec((F, F), lambda i: (0, 0)),
        pl.BlockSpec((F, F), lambda i: (0, 0)),
        pl.BlockSpec((1, F), lambda i: (0, 0)),
        pl.BlockSpec((1, F), lambda i: (0, 0)),
        pl.BlockSpec((NW, NB, BN), lambda i: (0, 0, 0)),
        pl.BlockSpec((NW, NB, BN), lambda i: (0, 0, 0)),
    ],
    out_specs=[
        pl.BlockSpec((BN, F), lambda i: (i, 0)),
        pl.BlockSpec((BN, F), lambda i: (i, 0)),
        pl.BlockSpec((NB, BN), lambda i: (0, 0)),
    ],
    out_shape=[
        jax.ShapeDtypeStruct((N, F), jnp.float32),
        jax.ShapeDtypeStruct((N, F), jnp.float32),
        jax.ShapeDtypeStruct((NB, BN), jnp.float32),
    ],
)


def _tc_edge_body(gl_ref, gr_ref, ea_ref, va_ref, we_ref, att_ref,
                  wr_ref, ex_ref):
    gl = gl_ref[...]
    m = gl + gr_ref[...] + ea_ref[...] * we_ref[...]
    m = jnp.maximum(m, 0.2 * m)
    e = jnp.sum(m * att_ref[...], axis=1, keepdims=True)
    ex = va_ref[...] * jnp.exp(e)
    ex_ref[...] = ex
    wr_ref[...] = gl * ex


_k3 = pl.pallas_call(
    _tc_edge_body,
    grid=(GE,),
    in_specs=[
        pl.BlockSpec((BE, F), lambda i: (i, 0)),
        pl.BlockSpec((BE, F), lambda i: (i, 0)),
        pl.BlockSpec((BE, 1), lambda i: (i, 0)),
        pl.BlockSpec((BE, 1), lambda i: (i, 0)),
        pl.BlockSpec((1, F), lambda i: (0, 0)),
        pl.BlockSpec((1, F), lambda i: (0, 0)),
    ],
    out_specs=[
        pl.BlockSpec((BE, F), lambda i: (i, 0)),
        pl.BlockSpec((BE, 1), lambda i: (i, 0)),
    ],
    out_shape=[
        jax.ShapeDtypeStruct((EP, F), jnp.float32),
        jax.ShapeDtypeStruct((EP, 1), jnp.float32),
    ],
)


def _tc_den_body(dp_ref, out_ref):
    out_ref[...] = jnp.sum(dp_ref[...], axis=0)


_k5a = pl.pallas_call(
    _tc_den_body,
    out_shape=jax.ShapeDtypeStruct((NB, BN), jnp.float32),
)


def _tc_comb_body(np_ref, den_ref, b_ref, wlt_ref, wrt_ref, bl_ref, br_ref,
                  xl_ref, xr_ref):
    ns = np_ref[0] + np_ref[1]
    h = ns / den_ref[...] + b_ref[...]
    h = jnp.maximum(h, 0.0)
    xl_ref[...] = _dot(h, wlt_ref[...]) + bl_ref[...]
    xr_ref[...] = _dot(h, wrt_ref[...]) + br_ref[...]


_k5 = pl.pallas_call(
    _tc_comb_body,
    grid=(NB,),
    in_specs=[
        pl.BlockSpec((NC, BN, F), lambda i: (0, i, 0)),
        pl.BlockSpec((BN, 1), lambda i: (i, 0)),
        pl.BlockSpec((1, F), lambda i: (0, 0)),
        pl.BlockSpec((F, F), lambda i: (0, 0)),
        pl.BlockSpec((F, F), lambda i: (0, 0)),
        pl.BlockSpec((1, F), lambda i: (0, 0)),
        pl.BlockSpec((1, F), lambda i: (0, 0)),
    ],
    out_specs=[
        pl.BlockSpec((BN, F), lambda i: (i, 0)),
        pl.BlockSpec((BN, F), lambda i: (i, 0)),
    ],
    out_shape=[
        jax.ShapeDtypeStruct((N, F), jnp.float32),
        jax.ShapeDtypeStruct((N, F), jnp.float32),
    ],
)


def _tc_final_body(np_ref, den_ref, b_ref, out_ref):
    out_ref[...] = (np_ref[0] + np_ref[1]) / den_ref[...] + b_ref[...]


_k5f = pl.pallas_call(
    _tc_final_body,
    grid=(NB,),
    in_specs=[
        pl.BlockSpec((NC, BN, F), lambda i: (0, i, 0)),
        pl.BlockSpec((BN, 1), lambda i: (i, 0)),
        pl.BlockSpec((1, F), lambda i: (0, 0)),
    ],
    out_specs=pl.BlockSpec((BN, F), lambda i: (i, 0)),
    out_shape=jax.ShapeDtypeStruct((N, F), jnp.float32),
)


def kernel(x, edge_index, edge_weight, Wl1, bl1, Wr1, br1, We1, att1, b1,
           Wl2, bl2, Wr2, br2, We2, att2, b2):
    src0 = edge_index[0]
    dst0 = edge_index[1]
    ew = edge_weight[:, 0]
    zn = jnp.zeros((N,), jnp.float32)
    zr = jnp.zeros((NP, F), jnp.float32)

    _k0, _k2, _k4, _kf = _sc_kernels()
    ea0, valid0, cnt_p, eas_p = _k0(src0, dst0, ew, zn)
    xl1, xr1, la = _k1(x, Wl1.T, Wr1.T, bl1[None], br1[None],
                       cnt_p.reshape(NW, NB, BN), eas_p.reshape(NW, NB, BN))

    loop_idx = jnp.arange(N, dtype=jnp.int32)
    pad = EP - E - N
    pad_idx = jnp.arange(pad, dtype=jnp.int32) % N
    srcp = jnp.concatenate([src0, loop_idx, pad_idx])
    dstp = jnp.concatenate([dst0, loop_idx, pad_idx])
    eap = jnp.concatenate([ea0, la.reshape(N), jnp.zeros((pad,), jnp.float32)])
    vap = jnp.concatenate([valid0, jnp.ones((N,), jnp.float32),
                           jnp.zeros((pad,), jnp.float32)])
    eap2 = eap[:, None]
    vap2 = vap[:, None]

    src3f = srcp.reshape(NW, CPWF, CHF)
    dst3f = dstp.reshape(NW, CPWF, CHF)

    def layer(xl, xr, wev, attv):
        nump, denp = _kf(xl, xr, src3f, dst3f, dstp, eap, vap,
                         wev, attv, zr, zn)
        den = _k5a(denp.reshape(NW, NB, BN)).reshape(N, 1)
        return nump.reshape(NC, NP, F), den

    nump1, den1 = layer(xl1, xr1, We1[:, 0], att1)
    xl2, xr2 = _k5(nump1, den1, b1[None], Wl2.T, Wr2.T, bl2[None], br2[None])
    nump2, den2 = layer(xl2, xr2, We2[:, 0], att2)
    return _k5f(nump2, den2, b2[None])


# speedup vs baseline: 1.3520x; 1.0870x over previous
import functools

import jax
import jax.numpy as jnp
from jax import lax
from jax.experimental import pallas as pl
from jax.experimental.pallas import tpu as pltpu
from jax.experimental.pallas import tpu_sc as plsc

N = 10000
E = 320000
F = 128

NC = 2
NS = 16
NW = NC * NS
L = 16

EP = 335872
CH = 128
EW = EP // NW
CPW = EW // CH

EPW0 = E // NW
CH0 = 80
CPW0 = EPW0 // CH0

NP = 10240
RPT = NP // NS

NB = 10
BN = N // NB
BE = 2048
GE = EP // BE

def _sc_preproc_body(src_hbm, dst_hbm, ew_hbm, zn_hbm,
                     ea0_hbm, valid_hbm, cnt_hbm, eas_hbm,
                     sall, dall, wall, eaall, vall, cnt_v, eas_v):
    c = lax.axis_index("c")
    s = lax.axis_index("s")
    w = c * NS + s
    b0 = w * EPW0
    pltpu.sync_copy(zn_hbm, cnt_v)
    pltpu.sync_copy(zn_hbm, eas_v)
    pltpu.sync_copy(src_hbm.at[pl.ds(b0, EPW0)], sall)
    pltpu.sync_copy(dst_hbm.at[pl.ds(b0, EPW0)], dall)
    pltpu.sync_copy(ew_hbm.at[pl.ds(b0, EPW0)], wall)

    UNROLL = 5

    def step(i, carry):
        for j in range(UNROLL):
            sl = pl.ds((i * UNROLL + j) * L, L)
            sv = sall[sl]
            dv = dall[sl]
            wv = wall[sl]
            mk = jnp.where(sv != dv, jnp.float32(1.0), jnp.float32(0.0))
            ea = mk * wv
            eaall[sl] = ea
            vall[sl] = mk
            plsc.addupdate_scatter(cnt_v, [dv], mk)
            plsc.addupdate_scatter(eas_v, [dv], ea)
        return carry

    lax.fori_loop(0, EPW0 // (L * UNROLL), step, 0)
    pltpu.sync_copy(eaall, ea0_hbm.at[pl.ds(b0, EPW0)])
    pltpu.sync_copy(vall, valid_hbm.at[pl.ds(b0, EPW0)])
    pltpu.sync_copy(cnt_v, cnt_hbm.at[pl.ds(w * N, N)])
    pltpu.sync_copy(eas_v, eas_hbm.at[pl.ds(w * N, N)])


@functools.lru_cache(maxsize=None)
def _sc_kernels():
    mesh = plsc.VectorSubcoreMesh(
        core_axis_name="c", subcore_axis_name="s",
        num_cores=NC, num_subcores=NS,
    )
    cparams = pltpu.CompilerParams(needs_layout_passes=False)
    k0 = pl.kernel(
        _sc_preproc_body,
        out_type=[
            jax.ShapeDtypeStruct((E,), jnp.float32),
            jax.ShapeDtypeStruct((E,), jnp.float32),
            jax.ShapeDtypeStruct((NW * N,), jnp.float32),
            jax.ShapeDtypeStruct((NW * N,), jnp.float32),
        ],
        mesh=mesh,
        compiler_params=cparams,
        scratch_types=[
            pltpu.VMEM((EPW0,), jnp.int32),
            pltpu.VMEM((EPW0,), jnp.int32),
            pltpu.VMEM((EPW0,), jnp.float32),
            pltpu.VMEM((EPW0,), jnp.float32),
            pltpu.VMEM((EPW0,), jnp.float32),
            pltpu.VMEM((N,), jnp.float32),
            pltpu.VMEM((N,), jnp.float32),
        ],
    )
    k2 = pl.kernel(
        _sc_gather_body,
        out_type=[
            jax.ShapeDtypeStruct((EP, F), jnp.float32),
            jax.ShapeDtypeStruct((EP, F), jnp.float32),
        ],
        mesh=mesh,
        compiler_params=cparams,
        scratch_types=[
            pltpu.VMEM((CH,), jnp.int32),
            pltpu.VMEM((CH,), jnp.int32),
            pltpu.VMEM((CH,), jnp.int32),
            pltpu.VMEM((CH,), jnp.int32),
            pltpu.VMEM((CH, F), jnp.float32),
            pltpu.VMEM((CH, F), jnp.float32),
            pltpu.VMEM((CH, F), jnp.float32),
            pltpu.VMEM((CH, F), jnp.float32),
            pltpu.SemaphoreType.DMA,
            pltpu.SemaphoreType.DMA,
            pltpu.SemaphoreType.DMA,
            pltpu.SemaphoreType.DMA,
            pltpu.SemaphoreType.DMA,
        ],
    )
    k4 = pl.kernel(
        _sc_scatter_body,
        out_type=[
            jax.ShapeDtypeStruct((NC * NP, F), jnp.float32),
            jax.ShapeDtypeStruct((NW * N,), jnp.float32),
        ],
        mesh=mesh,
        compiler_params=cparams,
        scratch_types=[
            pltpu.VMEM((2, CH), jnp.int32),
            pltpu.VMEM((2 * CH,), jnp.int32),
            pltpu.VMEM((2 * CH,), jnp.float32),
            pltpu.VMEM((CH, F), jnp.float32),
            pltpu.VMEM((CH, F), jnp.float32),
            pltpu.VMEM((N,), jnp.float32),
            pltpu.VMEM_SHARED((NP, F), jnp.float32),
            pltpu.SemaphoreType.DMA,
            pltpu.SemaphoreType.DMA,
        ],
    )
    kf = pl.kernel(
        _sc_layer_body,
        out_type=[
            jax.ShapeDtypeStruct((NC * NP, F), jnp.float32),
            jax.ShapeDtypeStruct((NW * N,), jnp.float32),
        ],
        mesh=mesh,
        compiler_params=cparams,
        scratch_types=[
            pltpu.VMEM((DEP, CHF), jnp.int32),
            pltpu.VMEM((DEP, CHF), jnp.int32),
            pltpu.VMEM((DEP * CHF,), jnp.int32),
            pltpu.VMEM((DEP * CHF,), jnp.float32),
            pltpu.VMEM((DEP * CHF,), jnp.float32),
            pltpu.VMEM((DEP * CHF,), jnp.float32),
            pltpu.VMEM((F,), jnp.float32),
            pltpu.VMEM((F,), jnp.float32),
            pltpu.VMEM((UNR * L,), jnp.float32),
            pltpu.VMEM((CHF, F), jnp.float32),
            pltpu.VMEM((CHF, F), jnp.float32),
            pltpu.VMEM((CHF, F), jnp.float32),
            pltpu.VMEM((CHF, F), jnp.float32),
            pltpu.VMEM((N,), jnp.float32),
            pltpu.VMEM_SHARED((NP, F), jnp.float32),
            pltpu.SemaphoreType.DMA,
            pltpu.SemaphoreType.DMA,
            pltpu.SemaphoreType.DMA,
            pltpu.SemaphoreType.DMA,
        ],
    )
    return k0, k2, k4, kf


DEP = 2


def _sc_gather_body(xl_hbm, xr_hbm, src_hbm, dst_hbm, gl_hbm, gr_hbm,
                    si0, di0, si1, di1, bl0, br0, bl1, br1,
                    is0, is1, gs0, gs1, wsem):
    c = lax.axis_index("c")
    s = lax.axis_index("s")
    w = c * NS + s
    b0 = w * EW
    bufs = ((si0, di0, bl0, br0, is0, gs0), (si1, di1, bl1, br1, is1, gs1))

    def it(t, carry):
        i0 = t * DEP
        icps = []
        for k in range(DEP):
            sik, dik, blk, brk, isk, gsk = bufs[k]
            base = b0 + (i0 + k) * CH
            icps.append((
                pltpu.async_copy(src_hbm.at[pl.ds(base, CH)], sik, isk),
                pltpu.async_copy(dst_hbm.at[pl.ds(base, CH)], dik, isk),
            ))
        gcps = []
        for k in range(DEP):
            sik, dik, blk, brk, isk, gsk = bufs[k]
            icps[k][0].wait()
            icps[k][1].wait()
            gcps.append((
                pltpu.async_copy(xl_hbm.at[sik], blk, gsk),
                pltpu.async_copy(xr_hbm.at[dik], brk, gsk),
            ))
        wcps = []
        for k in range(DEP):
            gcps[k][0].wait()
            gcps[k][1].wait()
            base = b0 + (i0 + k) * CH
            wcps.append(pltpu.async_copy(bufs[k][2], gl_hbm.at[pl.ds(base, CH)], wsem))
            wcps.append(pltpu.async_copy(bufs[k][3], gr_hbm.at[pl.ds(base, CH)], wsem))
        for cp in wcps:
            cp.wait()
        return carry

    lax.fori_loop(0, CPW // DEP, it, 0)


def _sc_scatter_body(wr_hbm, ex_hbm, dst_hbm, dst3_hbm, zr_hbm, zn_hbm,
                     nump_hbm, denp_hbm,
                     di2, di1, exb, bw0, bw1, den_v, num_s,
                     ls0, ls1):
    c = lax.axis_index("c")
    s = lax.axis_index("s")
    w = c * NS + s
    r0 = s * RPT
    b0 = w * EW
    pltpu.sync_copy(zn_hbm, den_v)
    pltpu.sync_copy(zr_hbm.at[pl.ds(r0, RPT)], num_s.at[pl.ds(r0, RPT)])
    plsc.subcore_barrier()
    bufs = ((bw0, ls0), (bw1, ls1))

    def it(t, carry):
        i0 = t * DEP
        lcps = []
        for k in range(DEP):
            bwk, sk = bufs[k]
            i = i0 + k
            base = b0 + i * CH
            lcps.append((
                pltpu.async_copy(wr_hbm.at[pl.ds(base, CH)], bwk, sk),
                pltpu.async_copy(dst3_hbm.at[w, i], di2.at[k], sk),
                pltpu.async_copy(dst_hbm.at[pl.ds(base, CH)],
                                 di1.at[pl.ds(k * CH, CH)], sk),
                pltpu.async_copy(ex_hbm.at[pl.ds(base, CH)],
                                 exb.at[pl.ds(k * CH, CH)], sk),
            ))
        for k in range(DEP):
            for cp in lcps[k]:
                cp.wait()
            pltpu.sync_copy(bufs[k][0], num_s.at[di2.at[k]], add=True)
            for j in range(CH // L):
                sl = pl.ds(k * CH + j * L, L)
                plsc.addupdate_scatter(den_v, [di1[sl]], exb[sl])
        return carry

    lax.fori_loop(0, CPW // DEP, it, 0)
    plsc.subcore_barrier()
    pltpu.sync_copy(den_v, denp_hbm.at[pl.ds(w * N, N)])
    pltpu.sync_copy(num_s.at[pl.ds(r0, RPT)],
                    nump_hbm.at[pl.ds(c * NP + r0, RPT)])


CHF = 64
UNR = 4
CPWF = EW // CHF


def _sc_layer_body(xl_hbm, xr_hbm, src3_hbm, dst3_hbm, dst_hbm, ea_hbm,
                   va_hbm, we_hbm, att_hbm, zr_hbm, zn_hbm,
                   nump_hbm, denp_hbm,
                   si2, di2, di1, eab, vab, exb, wev, attv, tmp,
                   bl0, br0, bl1, br1, den_v, num_s,
                   ls0, ls1, gs0, gs1):
    c = lax.axis_index("c")
    s = lax.axis_index("s")
    w = c * NS + s
    r0 = s * RPT
    b0 = w * EW
    pltpu.sync_copy(zn_hbm, den_v)
    pltpu.sync_copy(zr_hbm.at[pl.ds(r0, RPT)], num_s.at[pl.ds(r0, RPT)])
    pltpu.sync_copy(we_hbm, wev)
    pltpu.sync_copy(att_hbm, attv)
    plsc.subcore_barrier()
    lanes = lax.broadcasted_iota(jnp.int32, (L,), 0)
    bufs = ((bl0, br0, ls0, gs0), (bl1, br1, ls1, gs1))

    def it(t, carry):
        i0 = t * DEP
        lcps = []
        for k in range(DEP):
            blk, brk, lsk, gsk = bufs[k]
            i = i0 + k
            base = b0 + i * CHF
            lcps.append((
                pltpu.async_copy(src3_hbm.at[w, i], si2.at[k], lsk),
                pltpu.async_copy(dst3_hbm.at[w, i], di2.at[k], lsk),
                pltpu.async_copy(dst_hbm.at[pl.ds(base, CHF)],
                                 di1.at[pl.ds(k * CHF, CHF)], lsk),
                pltpu.async_copy(ea_hbm.at[pl.ds(base, CHF)],
                                 eab.at[pl.ds(k * CHF, CHF)], lsk),
                pltpu.async_copy(va_hbm.at[pl.ds(base, CHF)],
                                 vab.at[pl.ds(k * CHF, CHF)], lsk),
            ))
        gcps = []
        for k in range(DEP):
            blk, brk, lsk, gsk = bufs[k]
            for cp in lcps[k]:
                cp.wait()
            gcps.append((
                pltpu.async_copy(xl_hbm.at[si2.at[k]], blk, gsk),
                pltpu.async_copy(xr_hbm.at[di2.at[k]], brk, gsk),
            ))
        for k in range(DEP):
            blk, brk, lsk, gsk = bufs[k]
            gcps[k][0].wait()
            gcps[k][1].wait()

            kofs = jnp.full((L,), k * CHF, jnp.int32)
            l15 = jnp.full((L,), L - 1, jnp.int32)

            def edge(e2, carry2):
                evs = []
                for u in range(UNR):
                    e = e2 * UNR + u
                    easp = plsc.load_gather(eab, [kofs + e])
                    acc = jnp.zeros((L,), jnp.float32)
                    vals = []
                    for cslice in range(F // L):
                        sl = pl.ds(cslice * L, L)
                        bv = blk[e, sl]
                        vals.append(bv)
                        m = bv + brk[e, sl] + easp * wev[sl]
                        m = jnp.maximum(m, 0.2 * m)
                        acc = acc + attv[sl] * m
                    evs.append((e, acc, vals))
                for u in range(UNR):
                    tmp[pl.ds(u * L, L)] = plsc.cumsum(evs[u][1])
                for u in range(UNR):
                    e, acc, vals = evs[u]
                    esp = plsc.load_gather(tmp, [l15 + u * L])
                    vasp = plsc.load_gather(vab, [kofs + e])
                    ex = vasp * jnp.exp(esp)
                    plsc.store_scatter(exb, [kofs + e], ex)
                    for cslice in range(F // L):
                        sl = pl.ds(cslice * L, L)
                        blk[e, sl] = vals[cslice] * ex
                return carry2

            lax.fori_loop(0, CHF // UNR, edge, 0)
            pltpu.sync_copy(blk, num_s.at[di2.at[k]], add=True)
            for j in range(CHF // L):
                sl = pl.ds(k * CHF + j * L, L)
                plsc.addupdate_scatter(den_v, [di1[sl]], exb[sl])
        return carry

    lax.fori_loop(0, CPWF // DEP, it, 0)
    plsc.subcore_barrier()
    pltpu.sync_copy(den_v, denp_hbm.at[pl.ds(w * N, N)])
    pltpu.sync_copy(num_s.at[pl.ds(r0, RPT)],
                    nump_hbm.at[pl.ds(c * NP + r0, RPT)])


def _dot(a, b):
    return jnp.dot(a, b, preferred_element_type=jnp.float32,
                   precision=lax.Precision.HIGHEST)


def _tc_prep_body(x_ref, wlt_ref, wrt_ref, bl_ref, br_ref, cnt_ref, eas_ref,
                  xl_ref, xr_ref, la_ref):
    xb = x_ref[...]
    xl_ref[...] = _dot(xb, wlt_ref[...]) + bl_ref[...]
    xr_ref[...] = _dot(xb, wrt_ref[...]) + br_ref[...]

    @pl.when(pl.program_id(0) == 0)
    def _():
        cnt = jnp.sum(cnt_ref[...], axis=0)
        eas = jnp.sum(eas_ref[...], axis=0)
        la_ref[...] = eas / jnp.maximum(cnt, 1.0)


_k1 = pl.pallas_call(
    _tc_prep_body,
    grid=(NB,),
    in_specs=[
        pl.BlockSpec((BN, F), lambda i: (i, 0)),
        pl.BlockSpec((F, F), lambda i: (0, 0)),
        pl.BlockSpec((F, F), lambda i: (0, 0)),
        pl.BlockSpec((1, F), lambda i: (0, 0)),
        pl.BlockSpec((1, F), lambda i: (0, 0)),
        pl.BlockSpec((NW, NB, BN), lambda i: (0, 0, 0)),
        pl.BlockSpec((NW, NB, BN), lambda i: (0, 0, 0)),
    ],
    out_specs=[
        pl.BlockSpec((BN, F), lambda i: (i, 0)),
        pl.BlockSpec((BN, F), lambda i: (i, 0)),
        pl.BlockSpec((NB, BN), lambda i: (0, 0)),
    ],
    out_shape=[
        jax.ShapeDtypeStruct((N, F), jnp.float32),
        jax.ShapeDtypeStruct((N, F), jnp.float32),
        jax.ShapeDtypeStruct((NB, BN), jnp.float32),
    ],
)


def _tc_edge_body(gl_ref, gr_ref, ea_ref, va_ref, we_ref, att_ref,
                  wr_ref, ex_ref):
    gl = gl_ref[...]
    m = gl + gr_ref[...] + ea_ref[...] * we_ref[...]
    m = jnp.maximum(m, 0.2 * m)
    e = jnp.sum(m * att_ref[...], axis=1, keepdims=True)
    ex = va_ref[...] * jnp.exp(e)
    ex_ref[...] = ex
    wr_ref[...] = gl * ex


_k3 = pl.pallas_call(
    _tc_edge_body,
    grid=(GE,),
    in_specs=[
        pl.BlockSpec((BE, F), lambda i: (i, 0)),
        pl.BlockSpec((BE, F), lambda i: (i, 0)),
        pl.BlockSpec((BE, 1), lambda i: (i, 0)),
        pl.BlockSpec((BE, 1), lambda i: (i, 0)),
        pl.BlockSpec((1, F), lambda i: (0, 0)),
        pl.BlockSpec((1, F), lambda i: (0, 0)),
    ],
    out_specs=[
        pl.BlockSpec((BE, F), lambda i: (i, 0)),
        pl.BlockSpec((BE, 1), lambda i: (i, 0)),
    ],
    out_shape=[
        jax.ShapeDtypeStruct((EP, F), jnp.float32),
        jax.ShapeDtypeStruct((EP, 1), jnp.float32),
    ],
)


def _tc_den_body(dp_ref, out_ref):
    out_ref[...] = jnp.sum(dp_ref[...], axis=0)


_k5a = pl.pallas_call(
    _tc_den_body,
    out_shape=jax.ShapeDtypeStruct((NB, BN), jnp.float32),
)


def _tc_comb_body(np_ref, den_ref, b_ref, wlt_ref, wrt_ref, bl_ref, br_ref,
                  xl_ref, xr_ref):
    ns = np_ref[0] + np_ref[1]
    h = ns / den_ref[...] + b_ref[...]
    h = jnp.maximum(h, 0.0)
    xl_ref[...] = _dot(h, wlt_ref[...]) + bl_ref[...]
    xr_ref[...] = _dot(h, wrt_ref[...]) + br_ref[...]


_k5 = pl.pallas_call(
    _tc_comb_body,
    grid=(NB,),
    in_specs=[
        pl.BlockSpec((NC, BN, F), lambda i: (0, i, 0)),
        pl.BlockSpec((BN, 1), lambda i: (i, 0)),
        pl.BlockSpec((1, F), lambda i: (0, 0)),
        pl.BlockSpec((F, F), lambda i: (0, 0)),
        pl.BlockSpec((F, F), lambda i: (0, 0)),
        pl.BlockSpec((1, F), lambda i: (0, 0)),
        pl.BlockSpec((1, F), lambda i: (0, 0)),
    ],
    out_specs=[
        pl.BlockSpec((BN, F), lambda i: (i, 0)),
        pl.BlockSpec((BN, F), lambda i: (i, 0)),
    ],
    out_shape=[
        jax.ShapeDtypeStruct((N, F), jnp.float32),
        jax.ShapeDtypeStruct((N, F), jnp.float32),
    ],
)


def _tc_final_body(np_ref, den_ref, b_ref, out_ref):
    out_ref[...] = (np_ref[0] + np_ref[1]) / den_ref[...] + b_ref[...]


_k5f = pl.pallas_call(
    _tc_final_body,
    grid=(NB,),
    in_specs=[
        pl.BlockSpec((NC, BN, F), lambda i: (0, i, 0)),
        pl.BlockSpec((BN, 1), lambda i: (i, 0)),
        pl.BlockSpec((1, F), lambda i: (0, 0)),
    ],
    out_specs=pl.BlockSpec((BN, F), lambda i: (i, 0)),
    out_shape=jax.ShapeDtypeStruct((N, F), jnp.float32),
)


def kernel(x, edge_index, edge_weight, Wl1, bl1, Wr1, br1, We1, att1, b1,
           Wl2, bl2, Wr2, br2, We2, att2, b2):
    src0 = edge_index[0]
    dst0 = edge_index[1]
    ew = edge_weight[:, 0]
    zn = jnp.zeros((N,), jnp.float32)
    zr = jnp.zeros((NP, F), jnp.float32)

    _k0, _k2, _k4, _kf = _sc_kernels()
    ea0, valid0, cnt_p, eas_p = _k0(src0, dst0, ew, zn)
    xl1, xr1, la = _k1(x, Wl1.T, Wr1.T, bl1[None], br1[None],
                       cnt_p.reshape(NW, NB, BN), eas_p.reshape(NW, NB, BN))

    loop_idx = jnp.arange(N, dtype=jnp.int32)
    pad = EP - E - N
    pad_idx = jnp.arange(pad, dtype=jnp.int32) % N
    srcp = jnp.concatenate([src0, loop_idx, pad_idx])
    dstp = jnp.concatenate([dst0, loop_idx, pad_idx])
    eap = jnp.concatenate([ea0, la.reshape(N), jnp.zeros((pad,), jnp.float32)])
    vap = jnp.concatenate([valid0, jnp.ones((N,), jnp.float32),
                           jnp.zeros((pad,), jnp.float32)])
    eap2 = eap[:, None]
    vap2 = vap[:, None]

    src3f = srcp.reshape(NW, CPWF, CHF)
    dst3f = dstp.reshape(NW, CPWF, CHF)

    def layer(xl, xr, wev, attv):
        nump, denp = _kf(xl, xr, src3f, dst3f, dstp, eap, vap,
                         wev, attv, zr, zn)
        den = _k5a(denp.reshape(NW, NB, BN)).reshape(N, 1)
        return nump.reshape(NC, NP, F), den

    nump1, den1 = layer(xl1, xr1, We1[:, 0], att1)
    xl2, xr2 = _k5(nump1, den1, b1[None], Wl2.T, Wr2.T, bl2[None], br2[None])
    nump2, den2 = layer(xl2, xr2, We2[:, 0], att2)
    return _k5f(nump2, den2, b2[None])
